# Initial kernel scaffold; baseline (speedup 1.0000x reference)
import jax
import jax.numpy as jnp
from jax.experimental import pallas as pl


def kernel(obs, action, W1a, b1a, W2a, b2a, W1b, b1b, W2b, b2b, Wm, bm):
    raise NotImplementedError("write your pallas kernel here")


# speedup vs baseline: 2541.4975x; 2541.4975x over previous
import jax
import jax.numpy as jnp
from jax.experimental import pallas as pl

_NUM_NODES = 32
_GNN_OBS = 8
_GNN_ACT = 2
_TIN = 8


def _qcritic_kernel(obs_ref, act_ref, w1a_ref, b1a_ref, w2a_ref, b2a_ref,
                    w1b_ref, b1b_ref, w2b_ref, b2b_ref, wm_ref, bm_ref,
                    q1_ref, q2_ref):
    f32 = jnp.float32
    obs = obs_ref[:]
    act = act_ref[:]

    ro = jax.lax.broadcasted_iota(jnp.int32, (_NUM_NODES * _GNN_OBS, _TIN), 0)
    co = jax.lax.broadcasted_iota(jnp.int32, (_NUM_NODES * _GNN_OBS, _TIN), 1)
    sel_obs = jnp.where((ro % _GNN_OBS) == co + _GNN_ACT,
                        1.0 / _NUM_NODES, 0.0).astype(f32)
    ra = jax.lax.broadcasted_iota(jnp.int32, (_NUM_NODES * _GNN_ACT, _TIN), 0)
    ca = jax.lax.broadcasted_iota(jnp.int32, (_NUM_NODES * _GNN_ACT, _TIN), 1)
    sel_act = jnp.where((ra % _GNN_ACT) + (_TIN - _GNN_ACT) == ca,
                        1.0 / _NUM_NODES, 0.0).astype(f32)

    xm = (jnp.dot(obs, sel_obs, preferred_element_type=f32)
          + jnp.dot(act, sel_act, preferred_element_type=f32))

    wm = wm_ref[:]
    bm = bm_ref[:]

    def branch(w1, b1, w2, b2):
        h = jnp.maximum(jnp.dot(xm, w1, preferred_element_type=f32) + b1, 0.0)
        h = jnp.maximum(jnp.dot(h, w2, preferred_element_type=f32) + b2, 0.0)
        return jnp.dot(h, wm, preferred_element_type=f32) + bm

    s1 = branch(w1a_ref[:], b1a_ref[:], w2a_ref[:], b2a_ref[:])
    s2 = branch(w1b_ref[:], b1b_ref[:], w2b_ref[:], b2b_ref[:])
    q1_ref[:] = jnp.broadcast_to(s1, q1_ref.shape)
    q2_ref[:] = jnp.broadcast_to(s2, q2_ref.shape)


def kernel(obs, action, W1a, b1a, W2a, b2a, W1b, b1b, W2b, b2b, Wm, bm):
    bs = obs.shape[0]
    out_shape = [jax.ShapeDtypeStruct((bs, _NUM_NODES), jnp.float32),
                 jax.ShapeDtypeStruct((bs, _NUM_NODES), jnp.float32)]
    q1, q2 = pl.pallas_call(
        _qcritic_kernel,
        out_shape=out_shape,
    )(obs, action, W1a, b1a, W2a, b2a, W1b, b1b, W2b, b2b, Wm, bm)
    return (q1, q2)
